# Initial kernel scaffold; baseline (speedup 1.0000x reference)
import functools

import jax
import jax.numpy as jnp
from jax import lax
from jax.experimental import pallas as pl
from jax.experimental.pallas import tpu as pltpu
from jax.experimental.pallas import tpu_sc as plsc

N = 10000
E = 320000
ET = E + N
IN_CH = 128
C = 32
H = 4
NUM_CLASSES = 40
NEG = 0.2

NC, NS, L = 2, 16, 16
NW = NC * NS
CH = 128
KCH = 81
EPW = KCH * CH
ET_PAD = NW * EPW
NP = 10016
STRIDE = NP // NS
SW = 16

_mesh = plsc.VectorSubcoreMesh(
    core_axis_name="c", subcore_axis_name="s", num_cores=NC, num_subcores=NS)


def _f32(*shape):
    return jax.ShapeDtypeStruct(shape, jnp.float32)


def _dot(a, b):
    return lax.dot_general(a, b, (((1,), (0,)), ((), ())),
                           precision=lax.Precision.HIGHEST,
                           preferred_element_type=jnp.float32)


def _tc1_body(x_ref, wa_ref, wb_ref, oa_ref, ob_ref):
    i = pl.program_id(0)
    rows = i * 512 + lax.broadcasted_iota(jnp.int32, (512, 1), 0)
    mask = rows < N
    xv = x_ref[...]
    oa_ref[...] = jnp.where(mask, _dot(xv, wa_ref[...]), 0.0)
    ob_ref[...] = jnp.where(mask, _dot(xv, wb_ref[...]), 0.0)


def _tc1(x, wa, wb):
    return pl.pallas_call(
        _tc1_body,
        grid=(20,),
        in_specs=[pl.BlockSpec((512, IN_CH), lambda i: (i, 0)),
                  pl.BlockSpec((IN_CH, IN_CH), lambda i: (0, 0)),
                  pl.BlockSpec((IN_CH, IN_CH), lambda i: (0, 0))],
        out_specs=[pl.BlockSpec((512, IN_CH), lambda i: (i, 0)),
                   pl.BlockSpec((512, IN_CH), lambda i: (i, 0))],
        out_shape=[_f32(NP, IN_CH), _f32(NP, IN_CH)],
    )(x, wa, wb)


def _tc2_body(o0_ref, o1_ref, b_ref, wa_ref, wb_ref, oa_ref, ob_ref):
    i = pl.program_id(0)
    rows = i * 512 + lax.broadcasted_iota(jnp.int32, (512, 1), 0)
    mask = rows < N
    h = jnp.maximum(o0_ref[...] + o1_ref[...] + b_ref[...], 0.0)
    oa_ref[...] = jnp.where(mask, _dot(h, wa_ref[...]), 0.0)
    ob_ref[...] = jnp.where(mask, _dot(h, wb_ref[...]), 0.0)


def _tc2(o0, o1, b, wa, wb):
    return pl.pallas_call(
        _tc2_body,
        grid=(20,),
        in_specs=[pl.BlockSpec((512, IN_CH), lambda i: (i, 0)),
                  pl.BlockSpec((512, IN_CH), lambda i: (i, 0)),
                  pl.BlockSpec((1, IN_CH), lambda i: (0, 0)),
                  pl.BlockSpec((IN_CH, C), lambda i: (0, 0)),
                  pl.BlockSpec((IN_CH, C), lambda i: (0, 0))],
        out_specs=[pl.BlockSpec((512, C), lambda i: (i, 0)),
                   pl.BlockSpec((512, C), lambda i: (i, 0))],
        out_shape=[_f32(NP, C), _f32(NP, C)],
    )(o0, o1, b, wa, wb)


def _tc3_body(p0_ref, p1_ref, b_ref, w_ref, bl_ref, out_ref):
    h = jnp.maximum(p0_ref[...] + p1_ref[...] + b_ref[...], 0.0)
    out_ref[...] = _dot(h, w_ref[...]) + bl_ref[...]


def _tc3(p0, p1, b, w, bl):
    return pl.pallas_call(
        _tc3_body,
        grid=(25,),
        in_specs=[pl.BlockSpec((400, C), lambda i: (i, 0)),
                  pl.BlockSpec((400, C), lambda i: (i, 0)),
                  pl.BlockSpec((1, C), lambda i: (0, 0)),
                  pl.BlockSpec((C, NUM_CLASSES), lambda i: (0, 0)),
                  pl.BlockSpec((1, NUM_CLASSES), lambda i: (0, 0))],
        out_specs=pl.BlockSpec((400, NUM_CLASSES), lambda i: (i, 0)),
        out_shape=_f32(N, NUM_CLASSES),
    )(p0, p1, b, w, bl)


def _worker_id():
    return lax.axis_index("s") * NC + lax.axis_index("c")


def _full16(v):
    return jnp.full((L,), v, jnp.int32)


def _make_score_kernel(D, HH):
    CPH = D // HH

    def body(xl_hbm, xr_hbm, src_hbm, dst_hbm, att_hbm,
             scores_hbm, maxes_hbm,
             isrc, idst, xlb, xrb, sbuf, attv, attsp, maxb):
        wid = _worker_id()
        lanes = lax.iota(jnp.int32, L)
        pltpu.sync_copy(src_hbm.at[wid], isrc)
        pltpu.sync_copy(dst_hbm.at[wid], idst)
        pltpu.sync_copy(att_hbm, attv)
        for c in range(D):
            attsp[c] = plsc.load_gather(attv, [_full16(c)])

        def chunk(k, mcarry):
            pltpu.sync_copy(xl_hbm.at[isrc.at[k]], xlb)
            pltpu.sync_copy(xr_hbm.at[idst.at[k]], xrb)

            def group(g, mc):
                rows = lanes + g * L
                for h in range(HH):
                    acc = jnp.zeros((L,), jnp.float32)
                    for cc in range(CPH):
                        c = h * CPH + cc
                        a = plsc.load_gather(xlb, [rows, _full16(c)])
                        b = plsc.load_gather(xrb, [rows, _full16(c)])
                        v = a + b
                        m = jnp.maximum(v, v * NEG)
                        acc = acc + m * attsp[c]
                    sbuf[h, pl.ds(g * L, L)] = acc
                    mc = jnp.maximum(mc, acc)
                return mc

            mcarry = lax.fori_loop(0, CH // L, group, mcarry)
            pltpu.sync_copy(sbuf, scores_hbm.at[:, wid, k])
            return mcarry

        m = lax.fori_loop(0, KCH, chunk,
                          jnp.full((L,), -1e30, jnp.float32))
        maxb[...] = m
        pltpu.sync_copy(maxb, maxes_hbm.at[wid])

    return pl.kernel(
        body,
        out_type=[_f32(HH, NW, KCH, CH), _f32(NW, L)],
        mesh=_mesh,
        scratch_types=[
            pltpu.VMEM((KCH, CH), jnp.int32),
            pltpu.VMEM((KCH, CH), jnp.int32),
            pltpu.VMEM((CH, D), jnp.float32),
            pltpu.VMEM((CH, D), jnp.float32),
            pltpu.VMEM((HH, CH), jnp.float32),
            pltpu.VMEM((D,), jnp.float32),
            pltpu.VMEM((D, L), jnp.float32),
            pltpu.VMEM((L,), jnp.float32),
        ],
    )


def _make_expsum_kernel(HH):
    def body(scores_hbm, dst_hbm, m_hbm, z_hbm, sout_hbm,
             idst, sbuf, ebuf, mv, s_sh):
        cid = lax.axis_index("c")
        sid = lax.axis_index("s")
        wid = _worker_id()
        lanes = lax.iota(jnp.int32, L)
        pltpu.sync_copy(z_hbm.at[pl.ds(sid * STRIDE, STRIDE)],
                        s_sh.at[pl.ds(sid * STRIDE, STRIDE)])
        for r in range(CH):
            ebuf[r] = jnp.zeros((SW,), jnp.float32)
        plsc.subcore_barrier()
        pltpu.sync_copy(m_hbm, mv)
        pltpu.sync_copy(dst_hbm.at[wid], idst)
        mvec = mv[...]

        def chunk(k, _):
            pltpu.sync_copy(scores_hbm.at[:, wid, k], sbuf)

            def group(g, __):
                rows = lanes + g * L
                for h in range(HH):
                    e = jnp.exp(sbuf[h, pl.ds(g * L, L)] - mvec)
                    plsc.store_scatter(ebuf, [rows, _full16(h)], e)
                return 0

            lax.fori_loop(0, CH // L, group, 0)
            pltpu.sync_copy(ebuf, s_sh.at[idst.at[k]], add=True)
            return 0

        lax.fori_loop(0, KCH, chunk, 0)
        plsc.subcore_barrier()
        pltpu.sync_copy(s_sh.at[pl.ds(sid * STRIDE, STRIDE)],
                        sout_hbm.at[cid, pl.ds(sid * STRIDE, STRIDE)])

    return pl.kernel(
        body,
        out_type=_f32(NC, NP, SW),
        mesh=_mesh,
        scratch_types=[
            pltpu.VMEM((KCH, CH), jnp.int32),
            pltpu.VMEM((HH, CH), jnp.float32),
            pltpu.VMEM((CH, SW), jnp.float32),
            pltpu.VMEM((L,), jnp.float32),
            pltpu.VMEM_SHARED((NP, SW), jnp.float32),
        ],
    )


def _make_agg_kernel(D, HH):
    CPH = D // HH

    def body(xl_hbm, src_hbm, dst_hbm, scores_hbm, m_hbm, s0_hbm, s1_hbm,
             z_hbm, oout_hbm,
             isrc, idst, xlb, sbuf, s0b, s1b, mv, o_sh):
        cid = lax.axis_index("c")
        sid = lax.axis_index("s")
        wid = _worker_id()
        lanes = lax.iota(jnp.int32, L)
        pltpu.sync_copy(z_hbm.at[pl.ds(sid * STRIDE, STRIDE)],
                        o_sh.at[pl.ds(sid * STRIDE, STRIDE)])
        plsc.subcore_barrier()
        pltpu.sync_copy(m_hbm, mv)
        pltpu.sync_copy(src_hbm.at[wid], isrc)
        pltpu.sync_copy(dst_hbm.at[wid], idst)
        mvec = mv[...]

        def chunk(k, _):
            pltpu.sync_copy(xl_hbm.at[isrc.at[k]], xlb)
            pltpu.sync_copy(scores_hbm.at[:, wid, k], sbuf)
            pltpu.sync_copy(s0_hbm.at[idst.at[k]], s0b)
            pltpu.sync_copy(s1_hbm.at[idst.at[k]], s1b)

            def group(g, __):
                rows = lanes + g * L
                for h in range(HH):
                    e = jnp.exp(sbuf[h, pl.ds(g * L, L)] - mvec)
                    s0 = plsc.load_gather(s0b, [rows, _full16(h)])
                    s1 = plsc.load_gather(s1b, [rows, _full16(h)])
                    alpha = e / (s0 + s1 + 1e-16)
                    for cc in range(CPH):
                        c = h * CPH + cc
                        v = plsc.load_gather(xlb, [rows, _full16(c)])
                        plsc.store_scatter(xlb, [rows, _full16(c)], v * alpha)
                return 0

            lax.fori_loop(0, CH // L, group, 0)
            pltpu.sync_copy(xlb, o_sh.at[idst.at[k]], add=True)
            return 0

        lax.fori_loop(0, KCH, chunk, 0)
        plsc.subcore_barrier()
        pltpu.sync_copy(o_sh.at[pl.ds(sid * STRIDE, STRIDE)],
                        oout_hbm.at[cid, pl.ds(sid * STRIDE, STRIDE)])

    return pl.kernel(
        body,
        out_type=_f32(NC, NP, D),
        mesh=_mesh,
        scratch_types=[
            pltpu.VMEM((KCH, CH), jnp.int32),
            pltpu.VMEM((KCH, CH), jnp.int32),
            pltpu.VMEM((CH, D), jnp.float32),
            pltpu.VMEM((HH, CH), jnp.float32),
            pltpu.VMEM((CH, SW), jnp.float32),
            pltpu.VMEM((CH, SW), jnp.float32),
            pltpu.VMEM((L,), jnp.float32),
            pltpu.VMEM_SHARED((NP, D), jnp.float32),
        ],
    )


_score1 = _make_score_kernel(IN_CH, H)
_score2 = _make_score_kernel(C, 1)
_expsum1 = _make_expsum_kernel(H)
_expsum2 = _make_expsum_kernel(1)
_agg1 = _make_agg_kernel(IN_CH, H)
_agg2 = _make_agg_kernel(C, 1)


def kernel(x, edge_index, Wl1, Wr1, att1, b1, Wl2, Wr2, att2, b2, Wlin, blin):
    loop = jnp.arange(N, dtype=jnp.int32)
    src = jnp.concatenate([edge_index[0].astype(jnp.int32), loop,
                           jnp.zeros((ET_PAD - ET,), jnp.int32)])
    dst = jnp.concatenate([edge_index[1].astype(jnp.int32), loop,
                           jnp.full((ET_PAD - ET,), N, jnp.int32)])
    src = src.reshape(NW, KCH, CH)
    dst = dst.reshape(NW, KCH, CH)

    z_s = jnp.zeros((NP, SW), jnp.float32)
    z_o1 = jnp.zeros((NP, IN_CH), jnp.float32)
    z_o2 = jnp.zeros((NP, C), jnp.float32)

    xl1, xr1 = _tc1(x, Wl1, Wr1)
    sc1, mx1 = _score1(xl1, xr1, src, dst, att1.reshape(-1))
    m1 = jnp.full((L,), jnp.max(mx1), jnp.float32)
    s1 = _expsum1(sc1, dst, m1, z_s)
    o1 = _agg1(xl1, src, dst, sc1, m1, s1[0], s1[1], z_o1)

    xl2, xr2 = _tc2(o1[0], o1[1], b1.reshape(1, -1), Wl2, Wr2)
    sc2, mx2 = _score2(xl2, xr2, src, dst, att2.reshape(-1))
    m2 = jnp.full((L,), jnp.max(mx2), jnp.float32)
    s2 = _expsum2(sc2, dst, m2, z_s)
    o2 = _agg2(xl2, src, dst, sc2, m2, s2[0], s2[1], z_o2)

    return _tc3(o2[0], o2[1], b2.reshape(1, -1), Wlin, blin.reshape(1, -1))


# speedup vs baseline: 9.6162x; 9.6162x over previous
import functools

import jax
import jax.numpy as jnp
from jax import lax
from jax.experimental import pallas as pl
from jax.experimental.pallas import tpu as pltpu
from jax.experimental.pallas import tpu_sc as plsc

N = 10000
E = 320000
ET = E + N
IN_CH = 128
C = 32
H = 4
NUM_CLASSES = 40
NEG = 0.2

NC, NS, L = 2, 16, 16
NW = NC * NS
CH = 128
KCH = 81
EPW = KCH * CH
ET_PAD = NW * EPW
NP = 10112
STRIDE = NP // NS
SW = 16

_mesh = plsc.VectorSubcoreMesh(
    core_axis_name="c", subcore_axis_name="s", num_cores=NC, num_subcores=NS)


def _f32(*shape):
    return jax.ShapeDtypeStruct(shape, jnp.float32)


def _dot(a, b):
    return lax.dot_general(a, b, (((1,), (0,)), ((), ())),
                           precision=lax.Precision.HIGHEST,
                           preferred_element_type=jnp.float32)


def _tc1_body(x_ref, wa_ref, wb_ref, oa_ref, ob_ref):
    i = pl.program_id(0)
    rows = i * 512 + lax.broadcasted_iota(jnp.int32, (512, 1), 0)
    mask = rows < N
    xv = x_ref[...]
    oa_ref[...] = jnp.where(mask, _dot(xv, wa_ref[...]), 0.0)
    ob_ref[...] = jnp.where(mask, _dot(xv, wb_ref[...]), 0.0)


def _tc1(x, wa, wb):
    return pl.pallas_call(
        _tc1_body,
        grid=(20,),
        in_specs=[pl.BlockSpec((512, IN_CH), lambda i: (i, 0)),
                  pl.BlockSpec((IN_CH, IN_CH), lambda i: (0, 0)),
                  pl.BlockSpec((IN_CH, IN_CH), lambda i: (0, 0))],
        out_specs=[pl.BlockSpec((512, IN_CH), lambda i: (i, 0)),
                   pl.BlockSpec((512, IN_CH), lambda i: (i, 0))],
        out_shape=[_f32(NP, IN_CH), _f32(NP, IN_CH)],
    )(x, wa, wb)


def _tc2_body(o0_ref, o1_ref, b_ref, wa_ref, wb_ref, oa_ref, ob_ref):
    i = pl.program_id(0)
    rows = i * 512 + lax.broadcasted_iota(jnp.int32, (512, 1), 0)
    mask = rows < N
    h = jnp.maximum(o0_ref[...] + o1_ref[...] + b_ref[...], 0.0)
    oa_ref[...] = jnp.where(mask, _dot(h, wa_ref[...]), 0.0)
    ob_ref[...] = jnp.where(mask, _dot(h, wb_ref[...]), 0.0)


def _tc2(o0, o1, b, wa, wb):
    return pl.pallas_call(
        _tc2_body,
        grid=(20,),
        in_specs=[pl.BlockSpec((512, IN_CH), lambda i: (i, 0)),
                  pl.BlockSpec((512, IN_CH), lambda i: (i, 0)),
                  pl.BlockSpec((1, IN_CH), lambda i: (0, 0)),
                  pl.BlockSpec((IN_CH, C), lambda i: (0, 0)),
                  pl.BlockSpec((IN_CH, C), lambda i: (0, 0))],
        out_specs=[pl.BlockSpec((512, C), lambda i: (i, 0)),
                   pl.BlockSpec((512, C), lambda i: (i, 0))],
        out_shape=[_f32(NP, C), _f32(NP, C)],
    )(o0, o1, b, wa, wb)


def _tc3_body(p0_ref, p1_ref, b_ref, w_ref, bl_ref, out_ref):
    h = jnp.maximum(p0_ref[...] + p1_ref[...] + b_ref[...], 0.0)
    out_ref[...] = _dot(h, w_ref[...]) + bl_ref[...]


def _tc3(p0, p1, b, w, bl):
    return pl.pallas_call(
        _tc3_body,
        grid=(25,),
        in_specs=[pl.BlockSpec((400, C), lambda i: (i, 0)),
                  pl.BlockSpec((400, C), lambda i: (i, 0)),
                  pl.BlockSpec((1, C), lambda i: (0, 0)),
                  pl.BlockSpec((C, NUM_CLASSES), lambda i: (0, 0)),
                  pl.BlockSpec((1, NUM_CLASSES), lambda i: (0, 0))],
        out_specs=pl.BlockSpec((400, NUM_CLASSES), lambda i: (i, 0)),
        out_shape=_f32(N, NUM_CLASSES),
    )(p0, p1, b, w, bl)


def _worker_id():
    return lax.axis_index("s") * NC + lax.axis_index("c")


def _full16(v):
    return jnp.full((L,), v, jnp.int32)


def _make_score_kernel(D, HH):
    CPH = D // HH

    def body(xl_hbm, xr_hbm, src_hbm, dst_hbm, att_hbm,
             scores_hbm, maxes_hbm,
             isrc, idst, xlb, xrb, sbuf, attsp, maxb):
        wid = _worker_id()
        lanes = lax.iota(jnp.int32, L)
        pltpu.sync_copy(src_hbm.at[wid], isrc)
        pltpu.sync_copy(dst_hbm.at[wid], idst)
        pltpu.sync_copy(att_hbm, attsp)

        def chunk(k, mcarry):
            pltpu.sync_copy(xl_hbm.at[isrc.at[k]], xlb)
            pltpu.sync_copy(xr_hbm.at[idst.at[k]], xrb)

            def group(g, mc):
                rows = lanes + g * L
                for h in range(HH):
                    acc = jnp.zeros((L,), jnp.float32)
                    for cc in range(CPH):
                        c = h * CPH + cc
                        a = plsc.load_gather(xlb, [rows, _full16(c)])
                        b = plsc.load_gather(xrb, [rows, _full16(c)])
                        v = a + b
                        m = jnp.maximum(v, v * NEG)
                        acc = acc + m * attsp[c]
                    sbuf[h, pl.ds(g * L, L)] = acc
                    mc = jnp.maximum(mc, acc)
                return mc

            mcarry = lax.fori_loop(0, CH // L, group, mcarry)
            pltpu.sync_copy(sbuf, scores_hbm.at[:, wid, k])
            return mcarry

        m = lax.fori_loop(0, KCH, chunk,
                          jnp.full((L,), -1e30, jnp.float32))
        maxb[...] = m
        pltpu.sync_copy(maxb, maxes_hbm.at[wid])

    return pl.kernel(
        body,
        out_type=[_f32(HH, NW, KCH, CH), _f32(NW, L)],
        mesh=_mesh,
        compiler_params=pltpu.CompilerParams(needs_layout_passes=False, use_tc_tiling_on_sc=False),
        scratch_types=[
            pltpu.VMEM((KCH, CH), jnp.int32),
            pltpu.VMEM((KCH, CH), jnp.int32),
            pltpu.VMEM((CH, D), jnp.float32),
            pltpu.VMEM((CH, D), jnp.float32),
            pltpu.VMEM((HH, CH), jnp.float32),
            pltpu.VMEM((D, L), jnp.float32),
            pltpu.VMEM((L,), jnp.float32),
        ],
    )


def _make_expsum_kernel(HH):
    def body(scores_hbm, dst_hbm, m_hbm, z_hbm, sout_hbm,
             idst, sbuf, ebuf, mv, s_sh):
        cid = lax.axis_index("c")
        sid = lax.axis_index("s")
        wid = _worker_id()
        lanes = lax.iota(jnp.int32, L)
        pltpu.sync_copy(z_hbm.at[pl.ds(sid * STRIDE, STRIDE)],
                        s_sh.at[pl.ds(sid * STRIDE, STRIDE)])
        for r in range(CH):
            ebuf[r] = jnp.zeros((SW,), jnp.float32)
        plsc.subcore_barrier()
        pltpu.sync_copy(m_hbm, mv)
        pltpu.sync_copy(dst_hbm.at[wid], idst)
        mvec = mv[...]

        def chunk(k, _):
            pltpu.sync_copy(scores_hbm.at[:, wid, k], sbuf)

            def group(g, __):
                rows = lanes + g * L
                for h in range(HH):
                    e = jnp.exp(sbuf[h, pl.ds(g * L, L)] - mvec)
                    plsc.store_scatter(ebuf, [rows, _full16(h)], e)
                return 0

            lax.fori_loop(0, CH // L, group, 0)
            pltpu.sync_copy(ebuf, s_sh.at[idst.at[k]], add=True)
            return 0

        lax.fori_loop(0, KCH, chunk, 0)
        plsc.subcore_barrier()
        pltpu.sync_copy(s_sh.at[pl.ds(sid * STRIDE, STRIDE)],
                        sout_hbm.at[cid, pl.ds(sid * STRIDE, STRIDE)])

    return pl.kernel(
        body,
        out_type=_f32(NC, NP, SW),
        mesh=_mesh,
        compiler_params=pltpu.CompilerParams(needs_layout_passes=False, use_tc_tiling_on_sc=False),
        scratch_types=[
            pltpu.VMEM((KCH, CH), jnp.int32),
            pltpu.VMEM((HH, CH), jnp.float32),
            pltpu.VMEM((CH, SW), jnp.float32),
            pltpu.VMEM((L,), jnp.float32),
            pltpu.VMEM_SHARED((NP, SW), jnp.float32),
        ],
    )


def _make_agg_kernel(D, HH):
    CPH = D // HH

    def body(xl_hbm, src_hbm, dst_hbm, scores_hbm, m_hbm, s0_hbm, s1_hbm,
             z_hbm, oout_hbm,
             isrc, idst, xlb, sbuf, s0b, s1b, mv, o_sh):
        cid = lax.axis_index("c")
        sid = lax.axis_index("s")
        wid = _worker_id()
        lanes = lax.iota(jnp.int32, L)
        pltpu.sync_copy(z_hbm.at[pl.ds(sid * STRIDE, STRIDE)],
                        o_sh.at[pl.ds(sid * STRIDE, STRIDE)])
        plsc.subcore_barrier()
        pltpu.sync_copy(m_hbm, mv)
        pltpu.sync_copy(src_hbm.at[wid], isrc)
        pltpu.sync_copy(dst_hbm.at[wid], idst)
        mvec = mv[...]

        def chunk(k, _):
            pltpu.sync_copy(xl_hbm.at[isrc.at[k]], xlb)
            pltpu.sync_copy(scores_hbm.at[:, wid, k], sbuf)
            pltpu.sync_copy(s0_hbm.at[idst.at[k]], s0b)
            pltpu.sync_copy(s1_hbm.at[idst.at[k]], s1b)

            def group(g, __):
                rows = lanes + g * L
                for h in range(HH):
                    e = jnp.exp(sbuf[h, pl.ds(g * L, L)] - mvec)
                    s0 = plsc.load_gather(s0b, [rows, _full16(h)])
                    s1 = plsc.load_gather(s1b, [rows, _full16(h)])
                    alpha = e / (s0 + s1 + 1e-16)
                    for cc in range(CPH):
                        c = h * CPH + cc
                        v = plsc.load_gather(xlb, [rows, _full16(c)])
                        plsc.store_scatter(xlb, [rows, _full16(c)], v * alpha)
                return 0

            lax.fori_loop(0, CH // L, group, 0)
            pltpu.sync_copy(xlb, o_sh.at[idst.at[k]], add=True)
            return 0

        lax.fori_loop(0, KCH, chunk, 0)
        plsc.subcore_barrier()
        pltpu.sync_copy(o_sh.at[pl.ds(sid * STRIDE, STRIDE)],
                        oout_hbm.at[cid, pl.ds(sid * STRIDE, STRIDE)])

    return pl.kernel(
        body,
        out_type=_f32(NC, NP, D),
        mesh=_mesh,
        compiler_params=pltpu.CompilerParams(needs_layout_passes=False, use_tc_tiling_on_sc=False),
        scratch_types=[
            pltpu.VMEM((KCH, CH), jnp.int32),
            pltpu.VMEM((KCH, CH), jnp.int32),
            pltpu.VMEM((CH, D), jnp.float32),
            pltpu.VMEM((HH, CH), jnp.float32),
            pltpu.VMEM((CH, SW), jnp.float32),
            pltpu.VMEM((CH, SW), jnp.float32),
            pltpu.VMEM((L,), jnp.float32),
            pltpu.VMEM_SHARED((NP, D), jnp.float32),
        ],
    )


_score1 = _make_score_kernel(IN_CH, H)
_score2 = _make_score_kernel(C, 1)
_expsum1 = _make_expsum_kernel(H)
_expsum2 = _make_expsum_kernel(1)
_agg1 = _make_agg_kernel(IN_CH, H)
_agg2 = _make_agg_kernel(C, 1)


def kernel(x, edge_index, Wl1, Wr1, att1, b1, Wl2, Wr2, att2, b2, Wlin, blin):
    loop = jnp.arange(N, dtype=jnp.int32)
    src = jnp.concatenate([edge_index[0].astype(jnp.int32), loop,
                           jnp.zeros((ET_PAD - ET,), jnp.int32)])
    dst = jnp.concatenate([edge_index[1].astype(jnp.int32), loop,
                           jnp.full((ET_PAD - ET,), N, jnp.int32)])
    src = src.reshape(NW, KCH, CH)
    dst = dst.reshape(NW, KCH, CH)

    z_s = jnp.zeros((NP, SW), jnp.float32)
    z_o1 = jnp.zeros((NP, IN_CH), jnp.float32)
    z_o2 = jnp.zeros((NP, C), jnp.float32)

    xl1, xr1 = _tc1(x, Wl1, Wr1)
    sc1, mx1 = _score1(xl1, xr1, src, dst,
                       jnp.broadcast_to(att1.reshape(-1, 1), (IN_CH, L)))
    m1 = jnp.full((L,), jnp.max(mx1), jnp.float32)
    s1 = _expsum1(sc1, dst, m1, z_s)
    o1 = _agg1(xl1, src, dst, sc1, m1, s1[0], s1[1], z_o1)

    xl2, xr2 = _tc2(o1[0], o1[1], b1.reshape(1, -1), Wl2, Wr2)
    sc2, mx2 = _score2(xl2, xr2, src, dst,
                       jnp.broadcast_to(att2.reshape(-1, 1), (C, L)))
    m2 = jnp.full((L,), jnp.max(mx2), jnp.float32)
    s2 = _expsum2(sc2, dst, m2, z_s)
    o2 = _agg2(xl2, src, dst, sc2, m2, s2[0], s2[1], z_o2)

    return _tc3(o2[0], o2[1], b2.reshape(1, -1), Wlin, blin.reshape(1, -1))
